# Initial kernel scaffold; baseline (speedup 1.0000x reference)
import jax
import jax.numpy as jnp
from jax.experimental import pallas as pl


def kernel(x, table):
    raise NotImplementedError("write your pallas kernel here")


# speedup vs baseline: 1.2956x; 1.2956x over previous
import functools
import math

import jax
import jax.numpy as jnp
from jax import lax
from jax.experimental import pallas as pl
from jax.experimental.pallas import tpu as pltpu
from jax.experimental.pallas import tpu_sc as plsc

D_MODEL = 1024
SCALE = math.sqrt(float(D_MODEL))
LANES = 16
NUM_CORES = 2
NUM_SUBCORES = 16
NUM_WORKERS = NUM_CORES * NUM_SUBCORES

B_TOTAL = 4 * 2048
B_PER_W = B_TOTAL // NUM_WORKERS
CHUNK = 32
NCH = B_PER_W // CHUNK


def _sc_embed(table, x_flat):
    mesh = plsc.VectorSubcoreMesh(core_axis_name="c", subcore_axis_name="s")

    @functools.partial(
        pl.kernel,
        out_type=jax.ShapeDtypeStruct((B_TOTAL, D_MODEL), jnp.float32),
        mesh=mesh,
        scratch_types=[
            pltpu.VMEM((B_PER_W,), jnp.int32),
            pltpu.VMEM((CHUNK, D_MODEL), jnp.float32),
            pltpu.VMEM((CHUNK, D_MODEL), jnp.float32),
            pltpu.SemaphoreType.DMA,
            pltpu.SemaphoreType.DMA,
            pltpu.SemaphoreType.DMA,
            pltpu.SemaphoreType.DMA,
        ],
    )
    def k(table_hbm, x_hbm, out_hbm, idx_v, buf0, buf1, g0, g1, s0, s1):
        wid = lax.axis_index("s") * NUM_CORES + lax.axis_index("c")
        base = wid * B_PER_W
        pltpu.sync_copy(x_hbm.at[pl.ds(base, B_PER_W)], idx_v)

        bufs = (buf0, buf1)
        gsems = (g0, g1)
        ssems = (s0, s1)

        gathers = [None] * NCH
        stores = [None] * NCH

        def issue_gather(j):
            return pltpu.async_copy(
                table_hbm.at[idx_v.at[pl.ds(j * CHUNK, CHUNK)]],
                bufs[j & 1],
                gsems[j & 1],
            )

        gathers[0] = issue_gather(0)
        for j in range(NCH):
            cur = j & 1
            if j + 1 < NCH:
                if j >= 1:
                    stores[j - 1].wait()
                gathers[j + 1] = issue_gather(j + 1)
            gathers[j].wait()

            buf = bufs[cur]

            @pl.loop(0, CHUNK)
            def _(r):
                for c in range(0, D_MODEL, LANES):
                    sl = pl.ds(c, LANES)
                    buf.at[r, sl][...] = buf.at[r, sl][...] * SCALE

            stores[j] = pltpu.async_copy(
                buf,
                out_hbm.at[pl.ds(base + j * CHUNK, CHUNK)],
                ssems[cur],
            )
        stores[NCH - 2].wait()
        stores[NCH - 1].wait()

    return k(table, x_flat)


def kernel(x, table):
    x_flat = x.reshape(-1).astype(jnp.int32)
    out = _sc_embed(table, x_flat)
    return out.reshape(x.shape[0], x.shape[1], D_MODEL)


# speedup vs baseline: 1.3908x; 1.0735x over previous
import functools
import math

import jax
import jax.numpy as jnp
from jax import lax
from jax.experimental import pallas as pl
from jax.experimental.pallas import tpu as pltpu
from jax.experimental.pallas import tpu_sc as plsc

D_MODEL = 1024
SCALE = math.sqrt(float(D_MODEL))
LANES = 16
NUM_CORES = 2
NUM_SUBCORES = 16
NUM_WORKERS = NUM_CORES * NUM_SUBCORES

B_TOTAL = 4 * 2048
B_PER_W = B_TOTAL // NUM_WORKERS
CHUNK = 32
NCH = B_PER_W // CHUNK


def _sc_embed(table, x_flat):
    mesh = plsc.VectorSubcoreMesh(core_axis_name="c", subcore_axis_name="s")

    @functools.partial(
        pl.kernel,
        out_type=jax.ShapeDtypeStruct((B_TOTAL, D_MODEL), jnp.float32),
        mesh=mesh,
        scratch_types=[
            pltpu.VMEM((B_PER_W,), jnp.int32),
            pltpu.VMEM((CHUNK, D_MODEL), jnp.float32),
            pltpu.VMEM((CHUNK, D_MODEL), jnp.float32),
            pltpu.VMEM((CHUNK, D_MODEL), jnp.float32),
            pltpu.SemaphoreType.DMA,
            pltpu.SemaphoreType.DMA,
            pltpu.SemaphoreType.DMA,
            pltpu.SemaphoreType.DMA,
            pltpu.SemaphoreType.DMA,
            pltpu.SemaphoreType.DMA,
        ],
    )
    def k(table_hbm, x_hbm, out_hbm, idx_v,
          buf0, buf1, buf2, g0, g1, g2, s0, s1, s2):
        wid = lax.axis_index("s") * NUM_CORES + lax.axis_index("c")
        base = wid * B_PER_W
        pltpu.sync_copy(x_hbm.at[pl.ds(base, B_PER_W)], idx_v)

        bufs = (buf0, buf1, buf2)
        gsems = (g0, g1, g2)
        ssems = (s0, s1, s2)

        gathers = [None] * NCH
        stores = [None] * NCH

        def issue_gather(j):
            return pltpu.async_copy(
                table_hbm.at[idx_v.at[pl.ds(j * CHUNK, CHUNK)]],
                bufs[j % 3],
                gsems[j % 3],
            )

        gathers[0] = issue_gather(0)
        gathers[1] = issue_gather(1)
        for j in range(NCH):
            buf = bufs[j % 3]
            gathers[j].wait()

            @pl.loop(0, CHUNK)
            def _(r):
                for c in range(0, D_MODEL, LANES):
                    sl = pl.ds(c, LANES)
                    buf.at[r, sl][...] = buf.at[r, sl][...] * SCALE

            stores[j] = pltpu.async_copy(
                buf,
                out_hbm.at[pl.ds(base + j * CHUNK, CHUNK)],
                ssems[j % 3],
            )
            if j + 2 < NCH:
                if j >= 1:
                    stores[j - 1].wait()
                gathers[j + 2] = issue_gather(j + 2)
        stores[NCH - 3].wait()
        stores[NCH - 2].wait()
        stores[NCH - 1].wait()

    return k(table, x_flat)


def kernel(x, table):
    x_flat = x.reshape(-1).astype(jnp.int32)
    out = _sc_embed(table, x_flat)
    return out.reshape(x.shape[0], x.shape[1], D_MODEL)


# speedup vs baseline: 1.4277x; 1.0265x over previous
import functools
import math

import jax
import jax.numpy as jnp
from jax import lax
from jax.experimental import pallas as pl
from jax.experimental.pallas import tpu as pltpu
from jax.experimental.pallas import tpu_sc as plsc

D_MODEL = 1024
SCALE = math.sqrt(float(D_MODEL))
LANES = 16
NUM_CORES = 2
NUM_SUBCORES = 16
NUM_WORKERS = NUM_CORES * NUM_SUBCORES

B_TOTAL = 4 * 2048
B_PER_W = B_TOTAL // NUM_WORKERS
CHUNK = 16
NCH = B_PER_W // CHUNK
NBUF = 6


def _sc_embed(table, x_flat):
    mesh = plsc.VectorSubcoreMesh(core_axis_name="c", subcore_axis_name="s")

    @functools.partial(
        pl.kernel,
        out_type=jax.ShapeDtypeStruct((B_TOTAL, D_MODEL), jnp.float32),
        mesh=mesh,
        scratch_types=(
            [pltpu.VMEM((B_PER_W,), jnp.int32)]
            + [pltpu.VMEM((CHUNK, D_MODEL), jnp.float32)] * NBUF
            + [pltpu.SemaphoreType.DMA] * (2 * NBUF)
        ),
    )
    def k(table_hbm, x_hbm, out_hbm, idx_v, *scratch):
        bufs = scratch[:NBUF]
        gsems = scratch[NBUF:2 * NBUF]
        ssems = scratch[2 * NBUF:]
        wid = lax.axis_index("s") * NUM_CORES + lax.axis_index("c")
        base = wid * B_PER_W
        pltpu.sync_copy(x_hbm.at[pl.ds(base, B_PER_W)], idx_v)

        gathers = [None] * NCH
        stores = [None] * NCH

        def issue_gather(j):
            return pltpu.async_copy(
                table_hbm.at[idx_v.at[pl.ds(j * CHUNK, CHUNK)]],
                bufs[j % NBUF],
                gsems[j % NBUF],
            )

        for j in range(NBUF - 1):
            gathers[j] = issue_gather(j)
        for j in range(NCH):
            buf = bufs[j % NBUF]
            gathers[j].wait()

            @pl.loop(0, CHUNK)
            def _(r):
                for c in range(0, D_MODEL, LANES):
                    sl = pl.ds(c, LANES)
                    buf.at[r, sl][...] = buf.at[r, sl][...] * SCALE

            stores[j] = pltpu.async_copy(
                buf,
                out_hbm.at[pl.ds(base + j * CHUNK, CHUNK)],
                ssems[j % NBUF],
            )
            nxt = j + NBUF - 1
            if nxt < NCH:
                if nxt - NBUF >= 0:
                    stores[nxt - NBUF].wait()
                gathers[nxt] = issue_gather(nxt)
        for j in range(max(0, NCH - NBUF), NCH):
            stores[j].wait()

    return k(table, x_flat)


def kernel(x, table):
    x_flat = x.reshape(-1).astype(jnp.int32)
    out = _sc_embed(table, x_flat)
    return out.reshape(x.shape[0], x.shape[1], D_MODEL)
